# Initial kernel scaffold; baseline (speedup 1.0000x reference)
import jax
import jax.numpy as jnp
from jax.experimental import pallas as pl


def kernel(x, edge_index, Wn0, Ws0, b0, Wn1, Ws1, b1, Wn2, Ws2, b2):
    raise NotImplementedError("write your pallas kernel here")


# speedup vs baseline: 8.3353x; 8.3353x over previous
import functools

import jax
import jax.numpy as jnp
from jax import lax
from jax.experimental import pallas as pl
from jax.experimental.pallas import tpu as pltpu
from jax.experimental.pallas import tpu_sc as plsc

N = 10000
E = 320000
D = 128

NC = 2
NS = 16
NW = NC * NS
CHUNK = 128
NCHUNK = 80
EPW = CHUNK * NCHUNK
E_PAD = EPW * NW
N_PAD = 10240
ROWS_PER_TILE = N_PAD // NS
RB = ROWS_PER_TILE // CHUNK


_MESH = plsc.VectorSubcoreMesh(core_axis_name="c", subcore_axis_name="s")


def _make_agg():
    scratch = [
        pltpu.VMEM((NCHUNK, CHUNK), jnp.int32),
        pltpu.VMEM((NCHUNK, CHUNK), jnp.int32),
        pltpu.VMEM((CHUNK, D), jnp.float32),
        pltpu.VMEM_SHARED((N_PAD, D), jnp.float32),
        pltpu.SemaphoreType.DMA,
    ]

    def body(h_hbm, src_hbm, dst_hbm, out_hbm, src_v, dst_v, rows_v, acc,
             sem):
        c = lax.axis_index("c")
        s = lax.axis_index("s")
        wid = s * NC + c

        zv = jnp.zeros((16,), jnp.float32)
        def fill_body(i, _):
            r = i // 8
            cc = (i % 8) * 16
            rows_v[r, pl.ds(cc, 16)] = zv
            return 0
        lax.fori_loop(0, CHUNK * 8, fill_body, 0)

        pltpu.sync_copy(src_hbm.at[pl.ds(wid * NCHUNK, NCHUNK)], src_v)
        pltpu.sync_copy(dst_hbm.at[pl.ds(wid * NCHUNK, NCHUNK)], dst_v)

        def zero_body(j, _):
            r0 = s * ROWS_PER_TILE + j * CHUNK
            pltpu.sync_copy(rows_v, acc.at[pl.ds(r0, CHUNK)])
            return 0
        lax.fori_loop(0, RB, zero_body, 0)
        plsc.subcore_barrier()

        def edge_body(j, _):
            pltpu.async_copy(h_hbm.at[src_v.at[j]], rows_v, sem).wait()
            pltpu.sync_copy(rows_v, acc.at[dst_v.at[j]], add=True)
            return 0
        lax.fori_loop(0, NCHUNK, edge_body, 0)
        plsc.subcore_barrier()

        def rb_body(j, _):
            r0 = s * ROWS_PER_TILE + j * CHUNK
            pltpu.sync_copy(acc.at[pl.ds(r0, CHUNK)],
                            out_hbm.at[pl.ds(c * N_PAD + r0, CHUNK)])
            return 0
        lax.fori_loop(0, RB, rb_body, 0)

    return pl.kernel(body, mesh=_MESH,
                     out_type=jax.ShapeDtypeStruct((NC * N_PAD, D),
                                                   jnp.float32),
                     scratch_types=scratch)


def _make_cnt():
    scratch = [
        pltpu.VMEM((NCHUNK, CHUNK), jnp.int32),
        pltpu.VMEM((CHUNK,), jnp.float32),
        pltpu.VMEM((CHUNK,), jnp.float32),
        pltpu.VMEM_SHARED((N_PAD,), jnp.float32),
    ]

    def body(dst_hbm, cnt_hbm, dst_v, ones_v, z_v, cnt_acc):
        c = lax.axis_index("c")
        s = lax.axis_index("s")
        wid = s * NC + c

        zv = jnp.zeros((16,), jnp.float32)
        ov = jnp.ones((16,), jnp.float32)
        def fill_body(i, _):
            ones_v[pl.ds(i * 16, 16)] = ov
            z_v[pl.ds(i * 16, 16)] = zv
            return 0
        lax.fori_loop(0, CHUNK // 16, fill_body, 0)

        pltpu.sync_copy(dst_hbm.at[pl.ds(wid * NCHUNK, NCHUNK)], dst_v)

        def zero_body(j, _):
            r0 = s * ROWS_PER_TILE + j * CHUNK
            pltpu.sync_copy(z_v, cnt_acc.at[pl.ds(r0, CHUNK)])
            return 0
        lax.fori_loop(0, RB, zero_body, 0)
        plsc.subcore_barrier()

        def edge_body(j, _):
            pltpu.sync_copy(ones_v, cnt_acc.at[dst_v.at[j]], add=True)
            return 0
        lax.fori_loop(0, NCHUNK, edge_body, 0)
        plsc.subcore_barrier()

        def rb_body(j, _):
            r0 = s * ROWS_PER_TILE + j * CHUNK
            pltpu.sync_copy(cnt_acc.at[pl.ds(r0, CHUNK)],
                            cnt_hbm.at[pl.ds(c * N_PAD + r0, CHUNK)])
            return 0
        lax.fori_loop(0, RB, rb_body, 0)

    return pl.kernel(body, mesh=_MESH,
                     out_type=jax.ShapeDtypeStruct((NC * N_PAD,),
                                                   jnp.float32),
                     scratch_types=scratch)


_agg = _make_agg()
_cnt = _make_cnt()


def _tc_layer(p, cnt, h, Wn, Ws, b, relu):
    B = 512
    grid = (N_PAD // B,)

    def body(p0_ref, p1_ref, c0_ref, c1_ref, h_ref, wn_ref, ws_ref, b_ref,
             o_ref):
        cnt_col = c0_ref[...] + c1_ref[...]
        inv = 1.0 / jnp.maximum(cnt_col, 1.0)
        mean = (p0_ref[...] + p1_ref[...]) * inv
        acc = jnp.dot(mean, wn_ref[...], preferred_element_type=jnp.float32)
        acc = acc + jnp.dot(h_ref[...], ws_ref[...],
                            preferred_element_type=jnp.float32)
        acc = acc + b_ref[...]
        o_ref[...] = jnp.maximum(acc, 0.0) if relu else acc

    nb = N_PAD // B
    return pl.pallas_call(
        body,
        grid=grid,
        in_specs=[
            pl.BlockSpec((B, D), lambda i: (i, 0)),
            pl.BlockSpec((B, D), lambda i, _nb=nb: (i + _nb, 0)),
            pl.BlockSpec((B, 1), lambda i: (i, 0)),
            pl.BlockSpec((B, 1), lambda i, _nb=nb: (i + _nb, 0)),
            pl.BlockSpec((B, D), lambda i: (i, 0)),
            pl.BlockSpec((D, D), lambda i: (0, 0)),
            pl.BlockSpec((D, D), lambda i: (0, 0)),
            pl.BlockSpec((1, D), lambda i: (0, 0)),
        ],
        out_specs=pl.BlockSpec((B, D), lambda i: (i, 0)),
        out_shape=jax.ShapeDtypeStruct((N_PAD, D), jnp.float32),
    )(p, p, cnt.reshape(NC * N_PAD, 1), cnt.reshape(NC * N_PAD, 1), h, Wn,
      Ws, b.reshape(1, D))


def kernel(x, edge_index, Wn0, Ws0, b0, Wn1, Ws1, b1, Wn2, Ws2, b2):
    src = edge_index[0]
    dst = edge_index[1]
    pad = E_PAD - E
    pad_src = (jnp.arange(pad, dtype=jnp.int32) * 97) % N
    pad_dst = N + jnp.arange(pad, dtype=jnp.int32) % (N_PAD - N)
    src_p = jnp.concatenate([src, pad_src]).reshape(NW * NCHUNK, CHUNK)
    dst_p = jnp.concatenate([dst, pad_dst]).reshape(NW * NCHUNK, CHUNK)
    xp = jnp.pad(x, ((0, N_PAD - N), (0, 0)))

    cnt = _cnt(dst_p)
    p = _agg(xp, src_p, dst_p)
    h1 = _tc_layer(p, cnt, xp, Wn0, Ws0, b0, True)
    p = _agg(h1, src_p, dst_p)
    h2 = _tc_layer(p, cnt, h1, Wn1, Ws1, b1, True)
    p = _agg(h2, src_p, dst_p)
    h3 = _tc_layer(p, cnt, h2, Wn2, Ws2, b2, False)
    return h3[:N]


# speedup vs baseline: 10.7182x; 1.2859x over previous
import functools

import jax
import jax.numpy as jnp
from jax import lax
from jax.experimental import pallas as pl
from jax.experimental.pallas import tpu as pltpu
from jax.experimental.pallas import tpu_sc as plsc

N = 10000
E = 320000
D = 128

NC = 2
NS = 16
NW = NC * NS
CHUNK = 128
NCHUNK = 80
EPW = CHUNK * NCHUNK
E_PAD = EPW * NW
N_PAD = 10240
ROWS_PER_TILE = N_PAD // NS
RB = ROWS_PER_TILE // CHUNK


_MESH = plsc.VectorSubcoreMesh(core_axis_name="c", subcore_axis_name="s")


def _make_agg():
    scratch = [
        pltpu.VMEM((NCHUNK, CHUNK), jnp.int32),
        pltpu.VMEM((1, CHUNK), jnp.int32),
        pltpu.VMEM((1, CHUNK), jnp.int32),
        pltpu.VMEM((CHUNK, D), jnp.float32),
        pltpu.VMEM((CHUNK, D), jnp.float32),
        pltpu.VMEM_SHARED((N_PAD, D), jnp.float32),
        pltpu.SemaphoreType.DMA,
        pltpu.SemaphoreType.DMA,
        pltpu.SemaphoreType.DMA,
        pltpu.SemaphoreType.DMA,
    ]

    def body(h_hbm, src_hbm, dst_hbm, out_hbm, src_v, dst0, dst1,
             rows0, rows1, acc, g0, g1, d0, d1):
        c = lax.axis_index("c")
        s = lax.axis_index("s")
        wid = s * NC + c
        e0 = wid * NCHUNK

        zv = jnp.zeros((16,), jnp.float32)
        def fill_body(i, _):
            r = i // 8
            cc = (i % 8) * 16
            rows0[r, pl.ds(cc, 16)] = zv
            return 0
        lax.fori_loop(0, CHUNK * 8, fill_body, 0)

        pltpu.sync_copy(src_hbm.at[pl.ds(e0, NCHUNK)], src_v)

        def zero_body(j, _):
            r0 = s * ROWS_PER_TILE + j * CHUNK
            pltpu.sync_copy(rows0, acc.at[pl.ds(r0, CHUNK)])
            return 0
        lax.fori_loop(0, RB, zero_body, 0)
        plsc.subcore_barrier()

        pltpu.async_copy(h_hbm.at[src_v.at[0]], rows0, g0)
        pltpu.async_copy(dst_hbm.at[e0], dst0, d0)

        def pair_body(jp, _):
            j0 = 2 * jp
            pltpu.make_async_copy(h_hbm.at[src_v.at[j0]], rows0, g0).wait()
            pltpu.async_copy(h_hbm.at[src_v.at[j0 + 1]], rows1, g1)
            pltpu.make_async_copy(dst_hbm.at[e0 + j0],
                                  dst0, d0).wait()
            pltpu.async_copy(dst_hbm.at[e0 + j0 + 1], dst1, d1)
            pltpu.sync_copy(rows0, acc.at[dst0.at[0]], add=True)
            pltpu.make_async_copy(h_hbm.at[src_v.at[j0 + 1]],
                                  rows1, g1).wait()
            pltpu.async_copy(h_hbm.at[src_v.at[j0 + 2]], rows0, g0)
            pltpu.make_async_copy(dst_hbm.at[e0 + j0 + 1],
                                  dst1, d1).wait()
            pltpu.async_copy(dst_hbm.at[e0 + j0 + 2], dst0, d0)
            pltpu.sync_copy(rows1, acc.at[dst1.at[0]], add=True)
            return 0
        lax.fori_loop(0, NCHUNK // 2 - 1, pair_body, 0)

        jl = NCHUNK - 2
        pltpu.make_async_copy(h_hbm.at[src_v.at[jl]], rows0, g0).wait()
        pltpu.async_copy(h_hbm.at[src_v.at[jl + 1]], rows1, g1)
        pltpu.make_async_copy(dst_hbm.at[e0 + jl], dst0, d0).wait()
        pltpu.async_copy(dst_hbm.at[e0 + jl + 1], dst1, d1)
        pltpu.sync_copy(rows0, acc.at[dst0.at[0]], add=True)
        pltpu.make_async_copy(h_hbm.at[src_v.at[jl + 1]], rows1, g1).wait()
        pltpu.make_async_copy(dst_hbm.at[e0 + jl + 1],
                              dst1, d1).wait()
        pltpu.sync_copy(rows1, acc.at[dst1.at[0]], add=True)
        plsc.subcore_barrier()

        def rb_body(j, _):
            r0 = s * ROWS_PER_TILE + j * CHUNK
            pltpu.sync_copy(acc.at[pl.ds(r0, CHUNK)],
                            out_hbm.at[pl.ds(c * N_PAD + r0, CHUNK)])
            return 0
        lax.fori_loop(0, RB, rb_body, 0)

    return pl.kernel(body, mesh=_MESH,
                     out_type=jax.ShapeDtypeStruct((NC * N_PAD, D),
                                                   jnp.float32),
                     scratch_types=scratch)


def _make_cnt():
    scratch = [
        pltpu.VMEM((NCHUNK, CHUNK), jnp.int32),
        pltpu.VMEM((CHUNK,), jnp.float32),
        pltpu.VMEM((CHUNK,), jnp.float32),
        pltpu.VMEM_SHARED((N_PAD,), jnp.float32),
    ]

    def body(dst_hbm, cnt_hbm, dst_v, ones_v, z_v, cnt_acc):
        c = lax.axis_index("c")
        s = lax.axis_index("s")
        wid = s * NC + c

        zv = jnp.zeros((16,), jnp.float32)
        ov = jnp.ones((16,), jnp.float32)
        def fill_body(i, _):
            ones_v[pl.ds(i * 16, 16)] = ov
            z_v[pl.ds(i * 16, 16)] = zv
            return 0
        lax.fori_loop(0, CHUNK // 16, fill_body, 0)

        pltpu.sync_copy(dst_hbm.at[pl.ds(wid * NCHUNK, NCHUNK)], dst_v)

        def zero_body(j, _):
            r0 = s * ROWS_PER_TILE + j * CHUNK
            pltpu.sync_copy(z_v, cnt_acc.at[pl.ds(r0, CHUNK)])
            return 0
        lax.fori_loop(0, RB, zero_body, 0)
        plsc.subcore_barrier()

        def edge_body(j, _):
            pltpu.sync_copy(ones_v, cnt_acc.at[dst_v.at[j]], add=True)
            return 0
        lax.fori_loop(0, NCHUNK, edge_body, 0)
        plsc.subcore_barrier()

        def rb_body(j, _):
            r0 = s * ROWS_PER_TILE + j * CHUNK
            pltpu.sync_copy(cnt_acc.at[pl.ds(r0, CHUNK)],
                            cnt_hbm.at[pl.ds(c * N_PAD + r0, CHUNK)])
            return 0
        lax.fori_loop(0, RB, rb_body, 0)

    return pl.kernel(body, mesh=_MESH,
                     out_type=jax.ShapeDtypeStruct((NC * N_PAD,),
                                                   jnp.float32),
                     scratch_types=scratch)


_agg = _make_agg()
_cnt = _make_cnt()


def _tc_layer(p, cnt, h, Wn, Ws, b, relu):
    B = 512
    grid = (N_PAD // B,)

    def body(p0_ref, p1_ref, c0_ref, c1_ref, h_ref, wn_ref, ws_ref, b_ref,
             o_ref):
        cnt_col = c0_ref[...] + c1_ref[...]
        inv = 1.0 / jnp.maximum(cnt_col, 1.0)
        mean = (p0_ref[...] + p1_ref[...]) * inv
        acc = jnp.dot(mean, wn_ref[...], preferred_element_type=jnp.float32)
        acc = acc + jnp.dot(h_ref[...], ws_ref[...],
                            preferred_element_type=jnp.float32)
        acc = acc + b_ref[...]
        o_ref[...] = jnp.maximum(acc, 0.0) if relu else acc

    nb = N_PAD // B
    return pl.pallas_call(
        body,
        grid=grid,
        in_specs=[
            pl.BlockSpec((B, D), lambda i: (i, 0)),
            pl.BlockSpec((B, D), lambda i, _nb=nb: (i + _nb, 0)),
            pl.BlockSpec((B, 1), lambda i: (i, 0)),
            pl.BlockSpec((B, 1), lambda i, _nb=nb: (i + _nb, 0)),
            pl.BlockSpec((B, D), lambda i: (i, 0)),
            pl.BlockSpec((D, D), lambda i: (0, 0)),
            pl.BlockSpec((D, D), lambda i: (0, 0)),
            pl.BlockSpec((1, D), lambda i: (0, 0)),
        ],
        out_specs=pl.BlockSpec((B, D), lambda i: (i, 0)),
        out_shape=jax.ShapeDtypeStruct((N_PAD, D), jnp.float32),
    )(p, p, cnt.reshape(NC * N_PAD, 1), cnt.reshape(NC * N_PAD, 1), h, Wn,
      Ws, b.reshape(1, D))


def kernel(x, edge_index, Wn0, Ws0, b0, Wn1, Ws1, b1, Wn2, Ws2, b2):
    src = edge_index[0]
    dst = edge_index[1]
    pad = E_PAD - E
    pad_src = (jnp.arange(pad, dtype=jnp.int32) * 97) % N
    pad_dst = N + jnp.arange(pad, dtype=jnp.int32) % (N_PAD - N)
    src_p = jnp.concatenate([src, pad_src]).reshape(NW * NCHUNK, CHUNK)
    dst_p = jnp.concatenate([dst, pad_dst]).reshape(NW * NCHUNK, CHUNK)
    dst_p3 = dst_p.reshape(NW * NCHUNK, 1, CHUNK)
    xp = jnp.pad(x, ((0, N_PAD - N), (0, 0)))

    cnt = _cnt(dst_p)
    p = _agg(xp, src_p, dst_p3)
    h1 = _tc_layer(p, cnt, xp, Wn0, Ws0, b0, True)
    p = _agg(h1, src_p, dst_p3)
    h2 = _tc_layer(p, cnt, h1, Wn1, Ws1, b1, True)
    p = _agg(h2, src_p, dst_p3)
    h3 = _tc_layer(p, cnt, h2, Wn2, Ws2, b2, False)
    return h3[:N]
